# baseline (speedup 1.0000x reference)
import jax
import jax.numpy as jnp
from jax.experimental import pallas as pl
from jax.experimental.pallas import tpu as pltpu

_BIG = 1e11
_NSUM = 8


def _row_stats(f, state_row):
    s = f.shape[-1]
    lab = jax.lax.broadcasted_iota(jnp.int32, (_NSUM, s), 0)
    fb = jnp.broadcast_to(f, (_NSUM, s))
    eq = jnp.where(fb == lab + 1, 1.0, 0.0)
    nonpad = jnp.where(fb != 0, 1.0, 0.0)
    maskf = jnp.where(lab == 5, nonpad, eq)
    counts = jnp.sum(maskf, axis=1, keepdims=True)
    denom = jnp.where(counts > 0, counts, jnp.float32(_BIG))
    sums = jax.lax.dot_general(
        maskf, state_row,
        dimension_numbers=(((1,), (0,)), ((), ())),
        preferred_element_type=jnp.float32,
    )
    scaled = sums * (1.0 / denom)
    return scaled[:5], scaled[5:6]


def _body4(func_ref, state_ref, aug_ref, paper_ref):
    for i in range(func_ref.shape[1]):
        aug, paper = _row_stats(func_ref[0, i], state_ref[0, i])
        aug_ref[0, i] = aug
        paper_ref[0, i] = paper


def _body3(func_ref, state_ref, aug_ref, paper_ref):
    for i in range(func_ref.shape[0]):
        aug, paper = _row_stats(func_ref[i], state_ref[i])
        aug_ref[i] = aug
        paper_ref[i] = paper


def _segmean4(state4, func4, g):
    p, q, s, h = state4.shape
    qg = q // g
    assert q % g == 0

    def idx4(i):
        return (i // qg, i % qg, 0, 0)

    return pl.pallas_call(
        _body4,
        grid=(p * qg,),
        in_specs=[
            pl.BlockSpec((1, g, 1, s), idx4),
            pl.BlockSpec((1, g, s, h), idx4),
        ],
        out_specs=(
            pl.BlockSpec((1, g, 5, h), idx4),
            pl.BlockSpec((1, g, 1, h), idx4),
        ),
        out_shape=(
            jax.ShapeDtypeStruct((p, q, 5, h), jnp.float32),
            jax.ShapeDtypeStruct((p, q, 1, h), jnp.float32),
        ),
        compiler_params=pltpu.CompilerParams(
            dimension_semantics=("parallel",),
        ),
        name="segmean",
    )(func4, state4)


def _segmean3(state3, func3, g):
    r, s, h = state3.shape
    assert r % g == 0

    def idx3(i):
        return (i, 0, 0)

    return pl.pallas_call(
        _body3,
        grid=(r // g,),
        in_specs=[
            pl.BlockSpec((g, 1, s), idx3),
            pl.BlockSpec((g, s, h), idx3),
        ],
        out_specs=(
            pl.BlockSpec((g, 5, h), idx3),
            pl.BlockSpec((g, 1, h), idx3),
        ),
        out_shape=(
            jax.ShapeDtypeStruct((r, 5, h), jnp.float32),
            jax.ShapeDtypeStruct((r, 1, h), jnp.float32),
        ),
        compiler_params=pltpu.CompilerParams(
            dimension_semantics=("parallel",),
        ),
        name="segmean_tar",
    )(func3, state3)


def _masks(func):
    labels = jnp.arange(1, 6, dtype=func.dtype)
    aug_mask = jnp.any(func[..., :, None] == labels, axis=-2)
    return aug_mask, jnp.any(func != 0, axis=-1)


def kernel(tarsent_state, tar_func, refsent_state, ref_func):
    b, ts, h = tarsent_state.shape
    _, d, rs, _ = refsent_state.shape

    tar_aug, tarpaper = _segmean3(tarsent_state, tar_func.reshape(b, 1, ts), 2)
    ref_aug, refpaper = _segmean4(
        refsent_state, ref_func.reshape(b, d, 1, rs), 8)

    tar_aug_mask, tar_mask2 = _masks(tar_func)
    ref_aug_mask, ref_mask2 = _masks(ref_func)

    return (tar_aug, tar_aug_mask, ref_aug, ref_aug_mask,
            tarpaper.reshape(b, h), tar_mask2, refpaper.reshape(b, d, h),
            ref_mask2)


# speedup vs baseline: 1.1566x; 1.1566x over previous
import jax
import jax.numpy as jnp
from jax.experimental import pallas as pl
from jax.experimental.pallas import tpu as pltpu

_BIG = 1e11
_NSUM = 8


def _row_stats(f, state_row):
    s = f.shape[-1]
    lab = jax.lax.broadcasted_iota(jnp.int32, (_NSUM, s), 0)
    fb = jnp.broadcast_to(f, (_NSUM, s))
    eq = jnp.where(fb == lab + 1, 1.0, 0.0)
    nonpad = jnp.where(fb != 0, 1.0, 0.0)
    maskf = jnp.where(lab == 5, nonpad, eq)
    counts = jnp.sum(maskf, axis=1, keepdims=True)
    denom = jnp.where(counts > 0, counts, jnp.float32(_BIG))
    sums = jax.lax.dot_general(
        maskf, state_row,
        dimension_numbers=(((1,), (0,)), ((), ())),
        preferred_element_type=jnp.float32,
    )
    return sums * (1.0 / denom)


def _body_ref(func_ref, state_ref, aug_ref, paper_ref):
    g = func_ref.shape[1]
    for i in range(g):
        scaled = _row_stats(func_ref[0, i:i + 1], state_ref[0, i])
        aug_ref[0, :, i, :] = scaled[:5]
        paper_ref[0, 0, i:i + 1, :] = scaled[5:6]


def _body_tar(func_ref, state_ref, aug_ref, paper_ref):
    g = func_ref.shape[0]
    for i in range(g):
        scaled = _row_stats(func_ref[i:i + 1], state_ref[i])
        aug_ref[:, i, :] = scaled[:5]
        paper_ref[0, i:i + 1, :] = scaled[5:6]


def _segmean_ref(state4, func3, g):
    b, d, s, h = state4.shape
    dg = d // g
    assert d % g == 0

    def idx_in(i):
        return (i // dg, i % dg, 0, 0)

    def idx_out(i):
        return (i // dg, 0, i % dg, 0)

    return pl.pallas_call(
        _body_ref,
        grid=(b * dg,),
        in_specs=[
            pl.BlockSpec((1, g, s), lambda i: (i // dg, i % dg, 0)),
            pl.BlockSpec((1, g, s, h), idx_in),
        ],
        out_specs=(
            pl.BlockSpec((1, 5, g, h), idx_out),
            pl.BlockSpec((1, 1, g, h), idx_out),
        ),
        out_shape=(
            jax.ShapeDtypeStruct((b, 5, d, h), jnp.float32),
            jax.ShapeDtypeStruct((b, 1, d, h), jnp.float32),
        ),
        compiler_params=pltpu.CompilerParams(
            dimension_semantics=("parallel",),
        ),
        name="segmean_ref",
    )(func3, state4)


def _segmean_tar(state3, func2):
    b, s, h = state3.shape
    return pl.pallas_call(
        _body_tar,
        grid=(1,),
        in_specs=[
            pl.BlockSpec((b, s), lambda i: (0, 0)),
            pl.BlockSpec((b, s, h), lambda i: (0, 0, 0)),
        ],
        out_specs=(
            pl.BlockSpec((5, b, h), lambda i: (0, 0, 0)),
            pl.BlockSpec((1, b, h), lambda i: (0, 0, 0)),
        ),
        out_shape=(
            jax.ShapeDtypeStruct((5, b, h), jnp.float32),
            jax.ShapeDtypeStruct((1, b, h), jnp.float32),
        ),
        compiler_params=pltpu.CompilerParams(
            dimension_semantics=("arbitrary",),
        ),
        name="segmean_tar",
    )(func2, state3)


def _masks(func):
    labels = jnp.arange(1, 6, dtype=func.dtype)
    aug_mask = jnp.any(func[..., :, None] == labels, axis=-2)
    return aug_mask, jnp.any(func != 0, axis=-1)


def kernel(tarsent_state, tar_func, refsent_state, ref_func):
    b, ts, h = tarsent_state.shape
    _, d, rs, _ = refsent_state.shape

    tar_aug, tarpaper = _segmean_tar(tarsent_state, tar_func)
    ref_aug, refpaper = _segmean_ref(refsent_state, ref_func, 8)

    tar_aug_mask, tar_mask2 = _masks(tar_func)
    ref_aug_mask, ref_mask2 = _masks(ref_func)

    return (tar_aug.transpose(1, 0, 2), tar_aug_mask,
            ref_aug.transpose(0, 2, 1, 3), ref_aug_mask,
            tarpaper.reshape(b, h), tar_mask2,
            refpaper.transpose(0, 2, 1, 3).reshape(b, d, h), ref_mask2)


# speedup vs baseline: 1.2463x; 1.0775x over previous
import jax
import jax.numpy as jnp
from jax.experimental import pallas as pl
from jax.experimental.pallas import tpu as pltpu

_BIG = 1e11
_NSUM = 8


def _row_stats(f, state_row):
    s = f.shape[-1]
    lab = jax.lax.broadcasted_iota(jnp.int32, (_NSUM, s), 0)
    fb = jnp.broadcast_to(f, (_NSUM, s))
    eq = jnp.where(fb == lab + 1, 1.0, 0.0)
    nonpad = jnp.where(fb != 0, 1.0, 0.0)
    maskf = jnp.where(lab == 5, nonpad, eq)
    counts = jnp.sum(maskf, axis=1, keepdims=True)
    denom = jnp.where(counts > 0, counts, jnp.float32(_BIG))
    sums = jax.lax.dot_general(
        maskf, state_row,
        dimension_numbers=(((1,), (0,)), ((), ())),
        preferred_element_type=jnp.float32,
    )
    return sums * (1.0 / denom)


def _body_ref(func_ref, state_ref, aug_ref, paper_ref):
    g = func_ref.shape[1]
    for i in range(g):
        scaled = _row_stats(func_ref[0, i:i + 1], state_ref[0, i])
        aug_ref[0, :, i, :] = scaled[:5]
        paper_ref[0, 0, i:i + 1, :] = scaled[5:6]


def _body_tar(func_ref, state_ref, aug_ref, paper_ref):
    g = func_ref.shape[0]
    for i in range(g):
        scaled = _row_stats(func_ref[i:i + 1], state_ref[i])
        aug_ref[:, i, :] = scaled[:5]
        paper_ref[0, i:i + 1, :] = scaled[5:6]


def _segmean_ref(state4, func3, g):
    b, d, s, h = state4.shape
    dg = d // g
    assert d % g == 0

    def idx_in(i):
        return (i // dg, i % dg, 0, 0)

    def idx_out(i):
        return (i // dg, 0, i % dg, 0)

    return pl.pallas_call(
        _body_ref,
        grid=(b * dg,),
        in_specs=[
            pl.BlockSpec((1, g, s), lambda i: (i // dg, i % dg, 0)),
            pl.BlockSpec((1, g, s, h), idx_in),
        ],
        out_specs=(
            pl.BlockSpec((1, 5, g, h), idx_out),
            pl.BlockSpec((1, 1, g, h), idx_out),
        ),
        out_shape=(
            jax.ShapeDtypeStruct((b, 5, d, h), jnp.float32),
            jax.ShapeDtypeStruct((b, 1, d, h), jnp.float32),
        ),
        compiler_params=pltpu.CompilerParams(
            dimension_semantics=("parallel",),
        ),
        name="segmean_ref",
    )(func3, state4)


def _segmean_tar(state3, func2):
    b, s, h = state3.shape
    return pl.pallas_call(
        _body_tar,
        grid=(1,),
        in_specs=[
            pl.BlockSpec((b, s), lambda i: (0, 0)),
            pl.BlockSpec((b, s, h), lambda i: (0, 0, 0)),
        ],
        out_specs=(
            pl.BlockSpec((5, b, h), lambda i: (0, 0, 0)),
            pl.BlockSpec((1, b, h), lambda i: (0, 0, 0)),
        ),
        out_shape=(
            jax.ShapeDtypeStruct((5, b, h), jnp.float32),
            jax.ShapeDtypeStruct((1, b, h), jnp.float32),
        ),
        compiler_params=pltpu.CompilerParams(
            dimension_semantics=("arbitrary",),
        ),
        name="segmean_tar",
    )(func2, state3)


def _masks(func):
    labels = jnp.arange(1, 6, dtype=func.dtype)
    aug_mask = jnp.any(func[..., :, None] == labels, axis=-2)
    return aug_mask, jnp.any(func != 0, axis=-1)


def kernel(tarsent_state, tar_func, refsent_state, ref_func):
    b, ts, h = tarsent_state.shape
    _, d, rs, _ = refsent_state.shape

    tar_aug, tarpaper = _segmean_tar(tarsent_state, tar_func)
    ref_aug, refpaper = _segmean_ref(refsent_state, ref_func, 16)

    tar_aug_mask, tar_mask2 = _masks(tar_func)
    ref_aug_mask, ref_mask2 = _masks(ref_func)

    return (tar_aug.transpose(1, 0, 2), tar_aug_mask,
            ref_aug.transpose(0, 2, 1, 3), ref_aug_mask,
            tarpaper.reshape(b, h), tar_mask2,
            refpaper.transpose(0, 2, 1, 3).reshape(b, d, h), ref_mask2)


# speedup vs baseline: 1.2499x; 1.0029x over previous
import jax
import jax.numpy as jnp
from jax.experimental import pallas as pl
from jax.experimental.pallas import tpu as pltpu

_BIG = 1e11
_NSUM = 8


def _row_stats(f, state_row):
    s = f.shape[-1]
    lab = jax.lax.broadcasted_iota(jnp.int32, (_NSUM, s), 0)
    fb = jnp.broadcast_to(f, (_NSUM, s))
    eq = jnp.where(fb == lab + 1, 1.0, 0.0)
    nonpad = jnp.where(fb != 0, 1.0, 0.0)
    maskf = jnp.where(lab == 5, nonpad, eq)
    counts = jnp.sum(maskf, axis=1, keepdims=True)
    denom = jnp.where(counts > 0, counts, jnp.float32(_BIG))
    sums = jax.lax.dot_general(
        maskf, state_row,
        dimension_numbers=(((1,), (0,)), ((), ())),
        preferred_element_type=jnp.float32,
    )
    return sums * (1.0 / denom)


def _body_ref(func_ref, state_ref, aug_ref, paper_ref):
    g = func_ref.shape[1]
    for i in range(g):
        scaled = _row_stats(func_ref[0, i:i + 1], state_ref[0, i])
        aug_ref[0, :, i, :] = scaled[:5]
        paper_ref[0, 0, i:i + 1, :] = scaled[5:6]


def _body_tar(func_ref, state_ref, aug_ref, paper_ref):
    g = func_ref.shape[0]
    for i in range(g):
        scaled = _row_stats(func_ref[i:i + 1], state_ref[i])
        aug_ref[:, i, :] = scaled[:5]
        paper_ref[0, i:i + 1, :] = scaled[5:6]


def _segmean_ref(state4, func3, g):
    b, d, s, h = state4.shape
    dg = d // g
    assert d % g == 0

    def idx_in(i):
        return (i // dg, i % dg, 0, 0)

    def idx_out(i):
        return (i // dg, 0, i % dg, 0)

    return pl.pallas_call(
        _body_ref,
        grid=(b * dg,),
        in_specs=[
            pl.BlockSpec((1, g, s), lambda i: (i // dg, i % dg, 0)),
            pl.BlockSpec((1, g, s, h), idx_in),
        ],
        out_specs=(
            pl.BlockSpec((1, 5, g, h), idx_out),
            pl.BlockSpec((1, 1, g, h), idx_out),
        ),
        out_shape=(
            jax.ShapeDtypeStruct((b, 5, d, h), jnp.float32),
            jax.ShapeDtypeStruct((b, 1, d, h), jnp.float32),
        ),
        compiler_params=pltpu.CompilerParams(
            dimension_semantics=("parallel",),
            vmem_limit_bytes=56 * 1024 * 1024,
        ),
        name="segmean_ref",
    )(func3, state4)


def _segmean_tar(state3, func2):
    b, s, h = state3.shape
    return pl.pallas_call(
        _body_tar,
        grid=(1,),
        in_specs=[
            pl.BlockSpec((b, s), lambda i: (0, 0)),
            pl.BlockSpec((b, s, h), lambda i: (0, 0, 0)),
        ],
        out_specs=(
            pl.BlockSpec((5, b, h), lambda i: (0, 0, 0)),
            pl.BlockSpec((1, b, h), lambda i: (0, 0, 0)),
        ),
        out_shape=(
            jax.ShapeDtypeStruct((5, b, h), jnp.float32),
            jax.ShapeDtypeStruct((1, b, h), jnp.float32),
        ),
        compiler_params=pltpu.CompilerParams(
            dimension_semantics=("arbitrary",),
        ),
        name="segmean_tar",
    )(func2, state3)


def _masks(func):
    bits = jax.lax.reduce(1 << func, jnp.int32(0), jax.lax.bitwise_or,
                          (func.ndim - 1,))
    labels = jnp.arange(1, 6, dtype=jnp.int32)
    aug_mask = ((bits[..., None] >> labels) & 1) != 0
    return aug_mask, (bits & 0b111110) != 0


def kernel(tarsent_state, tar_func, refsent_state, ref_func):
    b, ts, h = tarsent_state.shape
    _, d, rs, _ = refsent_state.shape

    tar_aug, tarpaper = _segmean_tar(tarsent_state, tar_func)
    ref_aug, refpaper = _segmean_ref(refsent_state, ref_func, 32)

    tar_aug_mask, tar_mask2 = _masks(tar_func)
    ref_aug_mask, ref_mask2 = _masks(ref_func)

    return (tar_aug.transpose(1, 0, 2), tar_aug_mask,
            ref_aug.transpose(0, 2, 1, 3), ref_aug_mask,
            tarpaper.reshape(b, h), tar_mask2,
            refpaper.transpose(0, 2, 1, 3).reshape(b, d, h), ref_mask2)


# speedup vs baseline: 1.3078x; 1.0463x over previous
import jax
import jax.numpy as jnp
from jax.experimental import pallas as pl
from jax.experimental.pallas import tpu as pltpu

_BIG = 1e11
_NSUM = 8


def _row_stats(f, state_row):
    s = f.shape[-1]
    lab = jax.lax.broadcasted_iota(jnp.int32, (_NSUM, s), 0)
    fb = jnp.broadcast_to(f, (_NSUM, s))
    eq = jnp.where(fb == lab + 1, 1.0, 0.0)
    nonpad = jnp.where(fb != 0, 1.0, 0.0)
    maskf = jnp.where(lab == 5, nonpad, eq)
    counts = jnp.sum(maskf, axis=1, keepdims=True)
    denom = jnp.where(counts > 0, counts, jnp.float32(_BIG))
    sums = jax.lax.dot_general(
        maskf, state_row,
        dimension_numbers=(((1,), (0,)), ((), ())),
        preferred_element_type=jnp.float32,
    )
    return sums * (1.0 / denom)


def _body(ref_func_ref, ref_state_ref, tar_func_ref, tar_state_ref,
          raug_ref, rpaper_ref, taug_ref, tpaper_ref):
    d = ref_func_ref.shape[1]
    for i in range(d):
        scaled = _row_stats(ref_func_ref[0, i:i + 1], ref_state_ref[0, i])
        raug_ref[0, :, i, :] = scaled[:5]
        rpaper_ref[0, 0, i:i + 1, :] = scaled[5:6]

    @pl.when(pl.program_id(0) == 0)
    def _tar():
        b = tar_func_ref.shape[0]
        for i in range(b):
            scaled = _row_stats(tar_func_ref[i:i + 1], tar_state_ref[i])
            taug_ref[:, i, :] = scaled[:5]
            tpaper_ref[0, i:i + 1, :] = scaled[5:6]


def _segmean_all(tar_state, tar_func, ref_state, ref_func):
    b, d, s, h = ref_state.shape
    _, ts, _ = tar_state.shape
    raug, rpaper, taug, tpaper = pl.pallas_call(
        _body,
        grid=(b,),
        in_specs=[
            pl.BlockSpec((1, d, s), lambda i: (i, 0, 0)),
            pl.BlockSpec((1, d, s, h), lambda i: (i, 0, 0, 0)),
            pl.BlockSpec((b, ts), lambda i: (0, 0)),
            pl.BlockSpec((b, ts, h), lambda i: (0, 0, 0)),
        ],
        out_specs=(
            pl.BlockSpec((1, 5, d, h), lambda i: (i, 0, 0, 0)),
            pl.BlockSpec((1, 1, d, h), lambda i: (i, 0, 0, 0)),
            pl.BlockSpec((5, b, h), lambda i: (0, 0, 0)),
            pl.BlockSpec((1, b, h), lambda i: (0, 0, 0)),
        ),
        out_shape=(
            jax.ShapeDtypeStruct((b, 5, d, h), jnp.float32),
            jax.ShapeDtypeStruct((b, 1, d, h), jnp.float32),
            jax.ShapeDtypeStruct((5, b, h), jnp.float32),
            jax.ShapeDtypeStruct((1, b, h), jnp.float32),
        ),
        compiler_params=pltpu.CompilerParams(
            dimension_semantics=("arbitrary",),
            vmem_limit_bytes=56 * 1024 * 1024,
        ),
        name="segmean_all",
    )(ref_func, ref_state, tar_func, tar_state)
    return taug, tpaper, raug, rpaper


def _masks(func):
    bits = jax.lax.reduce(1 << func, jnp.int32(0), jax.lax.bitwise_or,
                          (func.ndim - 1,))
    labels = jnp.arange(1, 6, dtype=jnp.int32)
    aug_mask = ((bits[..., None] >> labels) & 1) != 0
    return aug_mask, (bits & 0b111110) != 0


def kernel(tarsent_state, tar_func, refsent_state, ref_func):
    b, ts, h = tarsent_state.shape
    _, d, rs, _ = refsent_state.shape

    tar_aug, tarpaper, ref_aug, refpaper = _segmean_all(
        tarsent_state, tar_func, refsent_state, ref_func)

    tar_aug_mask, tar_mask2 = _masks(tar_func)
    ref_aug_mask, ref_mask2 = _masks(ref_func)

    return (tar_aug.transpose(1, 0, 2), tar_aug_mask,
            ref_aug.transpose(0, 2, 1, 3), ref_aug_mask,
            tarpaper.reshape(b, h), tar_mask2,
            refpaper.transpose(0, 2, 1, 3).reshape(b, d, h), ref_mask2)
